# baseline (speedup 1.0000x reference)
import jax
import jax.numpy as jnp
from jax.experimental import pallas as pl
from jax.experimental.pallas import tpu as pltpu

_NUM_STATES = 1024
_VOCAB = 100000
_BATCH = 8
_BK = 64
_NK = _NUM_STATES // _BK
_CH = 2048
_NCH = (_VOCAB + _CH - 1) // _CH
_VPAD = _NCH * _CH


def _hmm_obs_kernel(z_ref, w_ref, b_ref, o_ref):
    k = pl.program_id(0)

    part = jnp.dot(z_ref[0], w_ref[...],
                   preferred_element_type=jnp.float32)

    @pl.when(k == 0)
    def _init():
        o_ref[:, pl.ds(0, _VOCAB)] = part + b_ref[...]

    @pl.when(k > 0)
    def _accum():
        o_ref[:, pl.ds(0, _VOCAB)] = o_ref[:, pl.ds(0, _VOCAB)] + part

    @pl.when(k == _NK - 1)
    def _normalize():
        iota = jax.lax.broadcasted_iota(jnp.int32, (1, _CH), 1)

        def stats_body(c, carry):
            m, s = carry
            x = o_ref[:, pl.ds(c * _CH, _CH)]
            x = jnp.where(c * _CH + iota < _VOCAB, x, -jnp.inf)
            bm = jnp.max(x, axis=-1, keepdims=True)
            m_new = jnp.maximum(m, bm)
            s_new = s * jnp.exp(m - m_new) + jnp.sum(
                jnp.exp(x - m_new), axis=-1, keepdims=True)
            return m_new, s_new

        m0 = jnp.full((_BATCH, 1), -jnp.inf, dtype=jnp.float32)
        s0 = jnp.zeros((_BATCH, 1), dtype=jnp.float32)
        m, s = jax.lax.fori_loop(0, _NCH, stats_body, (m0, s0))
        lse = m + jnp.log(s)

        def sub_body(c, carry):
            o_ref[:, pl.ds(c * _CH, _CH)] = (
                o_ref[:, pl.ds(c * _CH, _CH)] - lse)
            return carry

        jax.lax.fori_loop(0, _NCH, sub_body, 0)


def kernel(z, W_obs, b_obs):
    b2d = b_obs.reshape(1, _VOCAB)
    z3 = jnp.swapaxes(z.reshape(_BATCH, _NK, _BK), 0, 1)
    out = pl.pallas_call(
        _hmm_obs_kernel,
        grid=(_NK,),
        in_specs=[
            pl.BlockSpec((1, _BATCH, _BK), lambda k: (k, 0, 0)),
            pl.BlockSpec((_BK, _VOCAB), lambda k: (k, 0)),
            pl.BlockSpec((1, _VOCAB), lambda k: (0, 0)),
        ],
        out_specs=pl.BlockSpec((_BATCH, _VPAD), lambda k: (0, 0)),
        out_shape=jax.ShapeDtypeStruct((_BATCH, _VPAD), jnp.float32),
        compiler_params=pltpu.CompilerParams(
            dimension_semantics=("arbitrary",),
        ),
    )(z3, W_obs, b2d)
    return out[:, :_VOCAB]


# speedup vs baseline: 1.0368x; 1.0368x over previous
import jax
import jax.numpy as jnp
from jax.experimental import pallas as pl
from jax.experimental.pallas import tpu as pltpu

_NUM_STATES = 1024
_VOCAB = 100000
_BATCH = 8
_NS = 4
_BK = 16
_NG = _NUM_STATES // (_NS * _BK)
_NZ = _NUM_STATES // _BK
_CH = 2048
_NCH = (_VOCAB + _CH - 1) // _CH
_VPAD = _NCH * _CH


def _hmm_obs_kernel(z_ref, w0_ref, w1_ref, w2_ref, w3_ref, b_ref, o_ref):
    k = pl.program_id(0)
    w_refs = (w0_ref, w1_ref, w2_ref, w3_ref)

    part = jnp.dot(z_ref[k], w_refs[0][...],
                   preferred_element_type=jnp.float32)
    for i in range(1, _NS):
        part = part + jnp.dot(z_ref[i * _NG + k], w_refs[i][...],
                              preferred_element_type=jnp.float32)

    @pl.when(k == 0)
    def _init():
        o_ref[:, pl.ds(0, _VOCAB)] = part + b_ref[...]

    @pl.when(k > 0)
    def _accum():
        o_ref[:, pl.ds(0, _VOCAB)] = o_ref[:, pl.ds(0, _VOCAB)] + part

    @pl.when(k == _NG - 1)
    def _normalize():
        iota = jax.lax.broadcasted_iota(jnp.int32, (1, _CH), 1)

        def stats_body(c, carry):
            m, s = carry
            x = o_ref[:, pl.ds(c * _CH, _CH)]
            x = jnp.where(c * _CH + iota < _VOCAB, x, -jnp.inf)
            bm = jnp.max(x, axis=-1, keepdims=True)
            m_new = jnp.maximum(m, bm)
            s_new = s * jnp.exp(m - m_new) + jnp.sum(
                jnp.exp(x - m_new), axis=-1, keepdims=True)
            return m_new, s_new

        m0 = jnp.full((_BATCH, 1), -jnp.inf, dtype=jnp.float32)
        s0 = jnp.zeros((_BATCH, 1), dtype=jnp.float32)
        m, s = jax.lax.fori_loop(0, _NCH, stats_body, (m0, s0))
        lse = m + jnp.log(s)

        def sub_body(c, carry):
            o_ref[:, pl.ds(c * _CH, _CH)] = (
                o_ref[:, pl.ds(c * _CH, _CH)] - lse)
            return carry

        jax.lax.fori_loop(0, _NCH, sub_body, 0)


def _w_spec(i):
    return pl.BlockSpec((_BK, _VOCAB), lambda k, i=i: (i * _NG + k, 0))


def kernel(z, W_obs, b_obs):
    b2d = b_obs.reshape(1, _VOCAB)
    z3 = jnp.swapaxes(z.reshape(_BATCH, _NZ, _BK), 0, 1)
    out = pl.pallas_call(
        _hmm_obs_kernel,
        grid=(_NG,),
        in_specs=[
            pl.BlockSpec((_NZ, _BATCH, _BK), lambda k: (0, 0, 0)),
            _w_spec(0), _w_spec(1), _w_spec(2), _w_spec(3),
            pl.BlockSpec((1, _VOCAB), lambda k: (0, 0)),
        ],
        out_specs=pl.BlockSpec((_BATCH, _VPAD), lambda k: (0, 0)),
        out_shape=jax.ShapeDtypeStruct((_BATCH, _VPAD), jnp.float32),
        compiler_params=pltpu.CompilerParams(
            dimension_semantics=("arbitrary",),
        ),
    )(z3, W_obs, W_obs, W_obs, W_obs, b2d)
    return out[:, :_VOCAB]
